# Initial kernel scaffold; baseline (speedup 1.0000x reference)
import jax
import jax.numpy as jnp
from jax.experimental import pallas as pl


def kernel(x, edge_index, W1, b1, W2, b2, W3, b3, W4, b4, W5, b5, W6, b6, W7, b7, W8, b8, W9, b9, W10, b10):
    raise NotImplementedError("write your pallas kernel here")


# speedup vs baseline: 25.4517x; 25.4517x over previous
import jax
import jax.numpy as jnp
from jax import lax
from jax.experimental import pallas as pl
from jax.experimental.pallas import tpu as pltpu
from jax.experimental.pallas import tpu_sc as plsc

N = 10000
N_PAD = 10240
E = 320000
NC = 2
NS = 16
NW = NC * NS
EPW = E // NW
CH = 125
NCHUNK = EPW // CH
DEG_CH = 80
RPT = N_PAD // NS

_MESH = plsc.VectorSubcoreMesh(core_axis_name="c", subcore_axis_name="s")


def _zero_vmem_rows(buf, rows, width):
  def body(i, carry):
    for j in range(width // 16):
      buf[i, pl.ds(j * 16, 16)] = jnp.zeros((16,), jnp.float32)
    return carry
  lax.fori_loop(0, rows, body, 0)


def _make_agg_kernel(d):
  nbuf = 2 if d >= 128 else 4
  ngroup = NCHUNK // nbuf

  def body(g_hbm, ei_hbm, out_hbm, *rest):
    islots = rest[:nbuf]
    gbufs = rest[nbuf:2 * nbuf]
    acc = rest[2 * nbuf]
    isems = rest[2 * nbuf + 1:3 * nbuf + 1]
    gsems = rest[3 * nbuf + 1:]
    c = lax.axis_index("c")
    s = lax.axis_index("s")
    wid = s * NC + c
    base = wid * NCHUNK

    _zero_vmem_rows(gbufs[0], CH, d)
    for r in range(RPT // CH):
      pltpu.sync_copy(gbufs[0], acc.at[pl.ds(s * RPT + r * CH, CH)])
    rem = RPT - (RPT // CH) * CH
    if rem:
      pltpu.sync_copy(gbufs[0].at[pl.ds(0, rem)],
                      acc.at[pl.ds(s * RPT + (RPT // CH) * CH, rem)])
    plsc.subcore_barrier()

    for b in range(nbuf):
      pltpu.async_copy(ei_hbm.at[base + b], islots[b], isems[b])
    pltpu.make_async_copy(ei_hbm.at[base], islots[0], isems[0]).wait()
    pltpu.async_copy(g_hbm.at[islots[0].at[0]], gbufs[0], gsems[0])

    def group(gi, carry):
      for b in range(nbuf):
        k = gi * nbuf + b
        bn = (b + 1) % nbuf
        pltpu.make_async_copy(g_hbm.at[islots[b].at[0]], gbufs[b],
                              gsems[b]).wait()
        @pl.when(k + 1 < NCHUNK)
        def _():
          pltpu.make_async_copy(ei_hbm.at[base], islots[bn], isems[bn]).wait()
          pltpu.async_copy(g_hbm.at[islots[bn].at[0]], gbufs[bn], gsems[bn])
        pltpu.sync_copy(gbufs[b], acc.at[islots[b].at[1]], add=True)
        @pl.when(k + nbuf < NCHUNK)
        def _():
          pltpu.async_copy(ei_hbm.at[base + k + nbuf], islots[b], isems[b])
      return carry
    lax.fori_loop(0, ngroup, group, 0)

    plsc.subcore_barrier()
    pltpu.sync_copy(acc.at[pl.ds(s * RPT, RPT)],
                    out_hbm.at[pl.ds(c * N_PAD + s * RPT, RPT)])

  return pl.kernel(
      body,
      mesh=_MESH,
      compiler_params=pltpu.CompilerParams(use_tc_tiling_on_sc=False),
      out_type=jax.ShapeDtypeStruct((NC * N_PAD, d), jnp.float32),
      scratch_types=(
          [pltpu.VMEM((2, CH), jnp.int32) for _ in range(nbuf)]
          + [pltpu.VMEM((CH, d), jnp.float32) for _ in range(nbuf)]
          + [pltpu.VMEM_SHARED((N_PAD, d), jnp.float32)]
          + [pltpu.SemaphoreType.DMA for _ in range(2 * nbuf)]
      ),
  )


def _make_deg_kernel():
  def body(dst_hbm, out_hbm, dstv, ones, zb, acc):
    c = lax.axis_index("c")
    s = lax.axis_index("s")
    wid = s * NC + c

    pltpu.sync_copy(dst_hbm.at[wid], dstv)

    _zero_vmem_rows(zb, DEG_CH, 16)
    for r in range(RPT // DEG_CH):
      pltpu.sync_copy(zb, acc.at[pl.ds(s * RPT + r * DEG_CH, DEG_CH)])

    def setones(i, carry):
      ones[i, pl.ds(0, 16)] = jnp.ones((16,), jnp.float32)
      return carry
    lax.fori_loop(0, DEG_CH, setones, 0)
    plsc.subcore_barrier()

    def chunk(k, carry):
      pltpu.sync_copy(ones, acc.at[dstv.at[k]], add=True)
      return carry
    lax.fori_loop(0, EPW // DEG_CH, chunk, 0)

    plsc.subcore_barrier()
    pltpu.sync_copy(acc.at[pl.ds(s * RPT, RPT)],
                    out_hbm.at[pl.ds(c * N_PAD + s * RPT, RPT)])

  return pl.kernel(
      body,
      mesh=_MESH,
      compiler_params=pltpu.CompilerParams(use_tc_tiling_on_sc=False),
      out_type=jax.ShapeDtypeStruct((NC * N_PAD, 16), jnp.float32),
      scratch_types=[
          pltpu.VMEM((EPW // DEG_CH, DEG_CH), jnp.int32),
          pltpu.VMEM((DEG_CH, 16), jnp.float32),
          pltpu.VMEM((DEG_CH, 16), jnp.float32),
          pltpu.VMEM_SHARED((N_PAD, 16), jnp.float32),
      ],
  )


_agg_kernels = {}


def _agg(g, ei4):
  d = g.shape[1]
  if d not in _agg_kernels:
    _agg_kernels[d] = _make_agg_kernel(d)
  return _agg_kernels[d](g, ei4).reshape(NC, N_PAD, d)


_deg_kernel = []


def _deg(dst2):
  if not _deg_kernel:
    _deg_kernel.append(_make_deg_kernel())
  return _deg_kernel[0](dst2).reshape(NC, N_PAD, 16)


_TCB = 2000
_TCG = N // _TCB


def _tc0(x, degp):
  def body(x_ref, d0_ref, d1_ref, dinv_ref, gp_ref):
    deg = d0_ref[0][:, :1] + d1_ref[0][:, :1] + 1.0
    dinv = lax.rsqrt(deg)
    dinv_ref[...] = dinv
    gp_ref[...] = x_ref[...] * dinv

  din = x.shape[1]
  return pl.pallas_call(
      body,
      grid=(_TCG,),
      in_specs=[
          pl.BlockSpec((_TCB, din), lambda i: (i, 0)),
          pl.BlockSpec((1, _TCB, 16), lambda i: (0, i, 0)),
          pl.BlockSpec((1, _TCB, 16), lambda i: (1, i, 0)),
      ],
      out_specs=[
          pl.BlockSpec((_TCB, 1), lambda i: (i, 0)),
          pl.BlockSpec((_TCB, din), lambda i: (i, 0)),
      ],
      out_shape=[
          jax.ShapeDtypeStruct((N, 1), jnp.float32),
          jax.ShapeDtypeStruct((N, din), jnp.float32),
      ],
  )(x, degp, degp)


def _tc_layer(pp, gp, dinv, W, b, skip, agg_first, next_mode, next_w, want_x):
  da = gp.shape[1]
  dout = b.shape[0]
  b2 = b.reshape(1, dout)
  next_mm = next_mode == "mm"
  next_scale = next_mode == "scale"

  def body(*refs):
    idx = 0
    p0_ref = refs[idx]; idx += 1
    p1_ref = refs[idx]; idx += 1
    gp_ref = refs[idx]; idx += 1
    dinv_ref = refs[idx]; idx += 1
    if agg_first:
      w_ref = refs[idx]; idx += 1
    b_ref = refs[idx]; idx += 1
    if skip is not None:
      skip_ref = refs[idx]; idx += 1
    if next_mm:
      wn_ref = refs[idx]; idx += 1
    out_refs = refs[idx:]

    dinv_blk = dinv_ref[...]
    t = dinv_blk * (p0_ref[0] + p1_ref[0] + gp_ref[...])
    if agg_first:
      h = jnp.dot(t, w_ref[...], preferred_element_type=jnp.float32) + b_ref[...]
    else:
      h = t + b_ref[...]
    if skip is not None:
      h = h + skip_ref[...]
    x = jnp.maximum(h, 0.0)
    oi = 0
    if want_x:
      out_refs[oi][...] = x
      oi += 1
    if next_scale:
      out_refs[oi][...] = dinv_blk * x
    elif next_mm:
      out_refs[oi][...] = dinv_blk * jnp.dot(x, wn_ref[...],
                                             preferred_element_type=jnp.float32)

  in_specs = [
      pl.BlockSpec((1, _TCB, da), lambda i: (0, i, 0)),
      pl.BlockSpec((1, _TCB, da), lambda i: (1, i, 0)),
      pl.BlockSpec((_TCB, da), lambda i: (i, 0)),
      pl.BlockSpec((_TCB, 1), lambda i: (i, 0)),
  ]
  args = [pp, pp, gp, dinv]
  if agg_first:
    in_specs.append(pl.BlockSpec((da, dout), lambda i: (0, 0)))
    args.append(W)
  in_specs.append(pl.BlockSpec((1, dout), lambda i: (0, 0)))
  args.append(b2)
  if skip is not None:
    in_specs.append(pl.BlockSpec((_TCB, dout), lambda i: (i, 0)))
    args.append(skip)
  if next_mm:
    dnext = next_w.shape[1]
    in_specs.append(pl.BlockSpec((dout, dnext), lambda i: (0, 0)))
    args.append(next_w)

  out_specs = []
  out_shape = []
  if want_x:
    out_specs.append(pl.BlockSpec((_TCB, dout), lambda i: (i, 0)))
    out_shape.append(jax.ShapeDtypeStruct((N, dout), jnp.float32))
  if next_scale:
    out_specs.append(pl.BlockSpec((_TCB, dout), lambda i: (i, 0)))
    out_shape.append(jax.ShapeDtypeStruct((N, dout), jnp.float32))
  elif next_mm:
    dnext = next_w.shape[1]
    out_specs.append(pl.BlockSpec((_TCB, dnext), lambda i: (i, 0)))
    out_shape.append(jax.ShapeDtypeStruct((N, dnext), jnp.float32))

  res = pl.pallas_call(
      body,
      grid=(_TCG,),
      in_specs=in_specs,
      out_specs=out_specs,
      out_shape=out_shape,
  )(*args)
  x_out = res[0] if want_x else None
  gp_out = res[-1] if (next_scale or next_mm) else None
  return x_out, gp_out


DIMS = (128, 256, 128, 64, 32, 16, 32, 64, 128, 256, 128)
AGG_FIRST = tuple(DIMS[l - 1] <= DIMS[l] for l in range(1, 11))
SKIP_SRC = {6: 4, 7: 3, 8: 2, 9: 1}
WANT_X = {1, 2, 3, 4, 10}


def kernel(x, edge_index, W1, b1, W2, b2, W3, b3, W4, b4, W5, b5,
           W6, b6, W7, b7, W8, b8, W9, b9, W10, b10):
  Ws = (W1, W2, W3, W4, W5, W6, W7, W8, W9, W10)
  bs = (b1, b2, b3, b4, b5, b6, b7, b8, b9, b10)

  dst2 = edge_index[1].reshape(NW, EPW // DEG_CH, DEG_CH)
  ei4 = jnp.stack([edge_index[0].reshape(NW * NCHUNK, CH),
                   edge_index[1].reshape(NW * NCHUNK, CH)],
                  axis=1)

  degp = _deg(dst2)
  dinv, gp = _tc0(x, degp)

  saved = {}
  xl = None
  for l in range(1, 11):
    pp = _agg(gp, ei4)
    skip = saved.get(SKIP_SRC.get(l))
    if l < 10:
      next_mode = "scale" if AGG_FIRST[l] else "mm"
      next_w = None if AGG_FIRST[l] else Ws[l]
    else:
      next_mode, next_w = None, None
    xl, gp = _tc_layer(pp, gp, dinv,
                       Ws[l - 1] if AGG_FIRST[l - 1] else None,
                       bs[l - 1], skip, AGG_FIRST[l - 1], next_mode, next_w,
                       l in WANT_X)
    if l in SKIP_SRC.values():
      saved[l] = xl
  return xl


# speedup vs baseline: 32.9919x; 1.2963x over previous
import jax
import jax.numpy as jnp
from jax import lax
from jax.experimental import pallas as pl
from jax.experimental.pallas import tpu as pltpu
from jax.experimental.pallas import tpu_sc as plsc

N = 10000
N_PAD = 10240
E = 320000
NC = 2
NS = 16
NW = NC * NS
EPW = E // NW
CH = 125
NCHUNK = EPW // CH
DEG_CH = 625
RPT = N_PAD // NS

_MESH = plsc.VectorSubcoreMesh(core_axis_name="c", subcore_axis_name="s")


def _zero_vmem_rows(buf, rows, width):
  def body(i, carry):
    for j in range(width // 16):
      buf[i, pl.ds(j * 16, 16)] = jnp.zeros((16,), jnp.float32)
    return carry
  lax.fori_loop(0, rows, body, 0)


def _make_agg_kernel(d):
  ch = {128: 100, 64: 250, 32: 500, 16: 625}[d]
  nbuf = 2
  nchunk = EPW // ch

  def body(g_hbm, src_hbm, dst_hbm, out_hbm, *rest):
    srcv, dstv = rest[0], rest[1]
    gbufs = rest[2:2 + nbuf]
    acc = rest[2 + nbuf]
    gsems = rest[3 + nbuf:]
    c = lax.axis_index("c")
    s = lax.axis_index("s")
    wid = s * NC + c

    pltpu.sync_copy(src_hbm.at[wid], srcv)
    pltpu.sync_copy(dst_hbm.at[wid], dstv)

    _zero_vmem_rows(gbufs[0], min(ch, RPT), d)
    for r in range(RPT // ch):
      pltpu.sync_copy(gbufs[0], acc.at[pl.ds(s * RPT + r * ch, ch)])
    rem = RPT - (RPT // ch) * ch
    if rem:
      pltpu.sync_copy(gbufs[0].at[pl.ds(0, rem)],
                      acc.at[pl.ds(s * RPT + (RPT // ch) * ch, rem)])
    plsc.subcore_barrier()

    for b in range(nbuf):
      pltpu.async_copy(g_hbm.at[srcv.at[b]], gbufs[b], gsems[b])

    def group(gi, carry):
      for b in range(nbuf):
        k = gi * nbuf + b
        pltpu.make_async_copy(g_hbm.at[srcv.at[k]], gbufs[b],
                              gsems[b]).wait()
        pltpu.sync_copy(gbufs[b], acc.at[dstv.at[k]], add=True)
        @pl.when(k + nbuf < nchunk)
        def _():
          pltpu.async_copy(g_hbm.at[srcv.at[k + nbuf]], gbufs[b], gsems[b])
      return carry
    lax.fori_loop(0, nchunk // nbuf, group, 0)

    plsc.subcore_barrier()
    pltpu.sync_copy(acc.at[pl.ds(s * RPT, RPT)],
                    out_hbm.at[pl.ds(c * N_PAD + s * RPT, RPT)])

  return pl.kernel(
      body,
      mesh=_MESH,
      compiler_params=pltpu.CompilerParams(use_tc_tiling_on_sc=False),
      out_type=jax.ShapeDtypeStruct((NC * N_PAD, d), jnp.float32),
      scratch_types=(
          [pltpu.VMEM((nchunk, ch), jnp.int32),
           pltpu.VMEM((nchunk, ch), jnp.int32)]
          + [pltpu.VMEM((ch, d), jnp.float32) for _ in range(nbuf)]
          + [pltpu.VMEM_SHARED((N_PAD, d), jnp.float32)]
          + [pltpu.SemaphoreType.DMA for _ in range(nbuf)]
      ),
  ), ch


def _make_deg_kernel():
  def body(dst_hbm, out_hbm, dstv, ones, zb, acc):
    c = lax.axis_index("c")
    s = lax.axis_index("s")
    wid = s * NC + c

    pltpu.sync_copy(dst_hbm.at[wid], dstv)

    _zero_vmem_rows(zb, min(DEG_CH, RPT), 16)
    for r in range(RPT // DEG_CH):
      pltpu.sync_copy(zb, acc.at[pl.ds(s * RPT + r * DEG_CH, DEG_CH)])
    if RPT % DEG_CH:
      pltpu.sync_copy(zb.at[pl.ds(0, RPT % DEG_CH)],
                      acc.at[pl.ds(s * RPT + (RPT // DEG_CH) * DEG_CH,
                                   RPT % DEG_CH)])

    def setones(i, carry):
      ones[i, pl.ds(0, 16)] = jnp.ones((16,), jnp.float32)
      return carry
    lax.fori_loop(0, DEG_CH, setones, 0)
    plsc.subcore_barrier()

    def chunk(k, carry):
      pltpu.sync_copy(ones, acc.at[dstv.at[k]], add=True)
      return carry
    lax.fori_loop(0, EPW // DEG_CH, chunk, 0)

    plsc.subcore_barrier()
    pltpu.sync_copy(acc.at[pl.ds(s * RPT, RPT)],
                    out_hbm.at[pl.ds(c * N_PAD + s * RPT, RPT)])

  return pl.kernel(
      body,
      mesh=_MESH,
      compiler_params=pltpu.CompilerParams(use_tc_tiling_on_sc=False),
      out_type=jax.ShapeDtypeStruct((NC * N_PAD, 16), jnp.float32),
      scratch_types=[
          pltpu.VMEM((EPW // DEG_CH, DEG_CH), jnp.int32),
          pltpu.VMEM((DEG_CH, 16), jnp.float32),
          pltpu.VMEM((DEG_CH, 16), jnp.float32),
          pltpu.VMEM_SHARED((N_PAD, 16), jnp.float32),
      ],
  )


_agg_kernels = {}


def _agg(g, edge_index):
  d = g.shape[1]
  if d not in _agg_kernels:
    _agg_kernels[d] = _make_agg_kernel(d)
  k, ch = _agg_kernels[d]
  src3 = edge_index[0].reshape(NW, EPW // ch, ch)
  dst3 = edge_index[1].reshape(NW, EPW // ch, ch)
  return k(g, src3, dst3).reshape(NC, N_PAD, d)


_deg_kernel = []


def _deg(dst2):
  if not _deg_kernel:
    _deg_kernel.append(_make_deg_kernel())
  return _deg_kernel[0](dst2).reshape(NC, N_PAD, 16)


_TCB = 2000
_TCG = N // _TCB


def _tc0(x, degp):
  def body(x_ref, d0_ref, d1_ref, dinv_ref, gp_ref):
    deg = d0_ref[0][:, :1] + d1_ref[0][:, :1] + 1.0
    dinv = lax.rsqrt(deg)
    dinv_ref[...] = dinv
    gp_ref[...] = x_ref[...] * dinv

  din = x.shape[1]
  return pl.pallas_call(
      body,
      grid=(_TCG,),
      in_specs=[
          pl.BlockSpec((_TCB, din), lambda i: (i, 0)),
          pl.BlockSpec((1, _TCB, 16), lambda i: (0, i, 0)),
          pl.BlockSpec((1, _TCB, 16), lambda i: (1, i, 0)),
      ],
      out_specs=[
          pl.BlockSpec((_TCB, 1), lambda i: (i, 0)),
          pl.BlockSpec((_TCB, din), lambda i: (i, 0)),
      ],
      out_shape=[
          jax.ShapeDtypeStruct((N, 1), jnp.float32),
          jax.ShapeDtypeStruct((N, din), jnp.float32),
      ],
  )(x, degp, degp)


def _tc_layer(pp, gp, dinv, W, b, skip, agg_first, next_mode, next_w, want_x):
  da = gp.shape[1]
  dout = b.shape[0]
  b2 = b.reshape(1, dout)
  next_mm = next_mode == "mm"
  next_scale = next_mode == "scale"

  def body(*refs):
    idx = 0
    p0_ref = refs[idx]; idx += 1
    p1_ref = refs[idx]; idx += 1
    gp_ref = refs[idx]; idx += 1
    dinv_ref = refs[idx]; idx += 1
    if agg_first:
      w_ref = refs[idx]; idx += 1
    b_ref = refs[idx]; idx += 1
    if skip is not None:
      skip_ref = refs[idx]; idx += 1
    if next_mm:
      wn_ref = refs[idx]; idx += 1
    out_refs = refs[idx:]

    dinv_blk = dinv_ref[...]
    t = dinv_blk * (p0_ref[0] + p1_ref[0] + gp_ref[...])
    if agg_first:
      h = jnp.dot(t, w_ref[...], preferred_element_type=jnp.float32) + b_ref[...]
    else:
      h = t + b_ref[...]
    if skip is not None:
      h = h + skip_ref[...]
    x = jnp.maximum(h, 0.0)
    oi = 0
    if want_x:
      out_refs[oi][...] = x
      oi += 1
    if next_scale:
      out_refs[oi][...] = dinv_blk * x
    elif next_mm:
      out_refs[oi][...] = dinv_blk * jnp.dot(x, wn_ref[...],
                                             preferred_element_type=jnp.float32)

  in_specs = [
      pl.BlockSpec((1, _TCB, da), lambda i: (0, i, 0)),
      pl.BlockSpec((1, _TCB, da), lambda i: (1, i, 0)),
      pl.BlockSpec((_TCB, da), lambda i: (i, 0)),
      pl.BlockSpec((_TCB, 1), lambda i: (i, 0)),
  ]
  args = [pp, pp, gp, dinv]
  if agg_first:
    in_specs.append(pl.BlockSpec((da, dout), lambda i: (0, 0)))
    args.append(W)
  in_specs.append(pl.BlockSpec((1, dout), lambda i: (0, 0)))
  args.append(b2)
  if skip is not None:
    in_specs.append(pl.BlockSpec((_TCB, dout), lambda i: (i, 0)))
    args.append(skip)
  if next_mm:
    dnext = next_w.shape[1]
    in_specs.append(pl.BlockSpec((dout, dnext), lambda i: (0, 0)))
    args.append(next_w)

  out_specs = []
  out_shape = []
  if want_x:
    out_specs.append(pl.BlockSpec((_TCB, dout), lambda i: (i, 0)))
    out_shape.append(jax.ShapeDtypeStruct((N, dout), jnp.float32))
  if next_scale:
    out_specs.append(pl.BlockSpec((_TCB, dout), lambda i: (i, 0)))
    out_shape.append(jax.ShapeDtypeStruct((N, dout), jnp.float32))
  elif next_mm:
    dnext = next_w.shape[1]
    out_specs.append(pl.BlockSpec((_TCB, dnext), lambda i: (i, 0)))
    out_shape.append(jax.ShapeDtypeStruct((N, dnext), jnp.float32))

  res = pl.pallas_call(
      body,
      grid=(_TCG,),
      in_specs=in_specs,
      out_specs=out_specs,
      out_shape=out_shape,
  )(*args)
  x_out = res[0] if want_x else None
  gp_out = res[-1] if (next_scale or next_mm) else None
  return x_out, gp_out


DIMS = (128, 256, 128, 64, 32, 16, 32, 64, 128, 256, 128)
AGG_FIRST = tuple(DIMS[l - 1] <= DIMS[l] for l in range(1, 11))
SKIP_SRC = {6: 4, 7: 3, 8: 2, 9: 1}
WANT_X = {1, 2, 3, 4, 10}


def kernel(x, edge_index, W1, b1, W2, b2, W3, b3, W4, b4, W5, b5,
           W6, b6, W7, b7, W8, b8, W9, b9, W10, b10):
  Ws = (W1, W2, W3, W4, W5, W6, W7, W8, W9, W10)
  bs = (b1, b2, b3, b4, b5, b6, b7, b8, b9, b10)

  dst2 = edge_index[1].reshape(NW, EPW // DEG_CH, DEG_CH)

  degp = _deg(dst2)
  dinv, gp = _tc0(x, degp)

  saved = {}
  xl = None
  for l in range(1, 11):
    pp = _agg(gp, edge_index)
    skip = saved.get(SKIP_SRC.get(l))
    if l < 10:
      next_mode = "scale" if AGG_FIRST[l] else "mm"
      next_w = None if AGG_FIRST[l] else Ws[l]
    else:
      next_mode, next_w = None, None
    xl, gp = _tc_layer(pp, gp, dinv,
                       Ws[l - 1] if AGG_FIRST[l - 1] else None,
                       bs[l - 1], skip, AGG_FIRST[l - 1], next_mode, next_w,
                       l in WANT_X)
    if l in SKIP_SRC.values():
      saved[l] = xl
  return xl


# speedup vs baseline: 32.9941x; 1.0001x over previous
import jax
import jax.numpy as jnp
from jax import lax
from jax.experimental import pallas as pl
from jax.experimental.pallas import tpu as pltpu
from jax.experimental.pallas import tpu_sc as plsc

N = 10000
E = 320000
NC = 2
NS = 16
NW = NC * NS
EPW = E // NW
CH = 125
NCHUNK = EPW // CH
DEG_CH = 625
RPT = N // NS

_MESH = plsc.VectorSubcoreMesh(core_axis_name="c", subcore_axis_name="s")


def _zero_vmem_rows(buf, rows, width):
  def body(i, carry):
    for j in range(width // 16):
      buf[i, pl.ds(j * 16, 16)] = jnp.zeros((16,), jnp.float32)
    return carry
  lax.fori_loop(0, rows, body, 0)


def _make_agg_kernel(d):
  ch = {128: 100, 64: 250, 32: 500, 16: 625}[d]
  nbuf = 2
  nchunk = EPW // ch

  def body(g_hbm, src_hbm, dst_hbm, out_hbm, *rest):
    srcv, dstv = rest[0], rest[1]
    gbufs = rest[2:2 + nbuf]
    acc = rest[2 + nbuf]
    gsems = rest[3 + nbuf:]
    c = lax.axis_index("c")
    s = lax.axis_index("s")
    wid = s * NC + c

    pltpu.sync_copy(src_hbm.at[wid], srcv)
    pltpu.sync_copy(dst_hbm.at[wid], dstv)

    _zero_vmem_rows(gbufs[0], min(ch, RPT), d)
    for r in range(RPT // ch):
      pltpu.sync_copy(gbufs[0], acc.at[pl.ds(s * RPT + r * ch, ch)])
    rem = RPT - (RPT // ch) * ch
    if rem:
      pltpu.sync_copy(gbufs[0].at[pl.ds(0, rem)],
                      acc.at[pl.ds(s * RPT + (RPT // ch) * ch, rem)])
    plsc.subcore_barrier()

    for b in range(nbuf):
      pltpu.async_copy(g_hbm.at[srcv.at[b]], gbufs[b], gsems[b])

    def group(gi, carry):
      for b in range(nbuf):
        k = gi * nbuf + b
        pltpu.make_async_copy(g_hbm.at[srcv.at[k]], gbufs[b],
                              gsems[b]).wait()
        pltpu.sync_copy(gbufs[b], acc.at[dstv.at[k]], add=True)
        @pl.when(k + nbuf < nchunk)
        def _():
          pltpu.async_copy(g_hbm.at[srcv.at[k + nbuf]], gbufs[b], gsems[b])
      return carry
    lax.fori_loop(0, nchunk // nbuf, group, 0)

    plsc.subcore_barrier()
    pltpu.sync_copy(acc.at[pl.ds(s * RPT, RPT)],
                    out_hbm.at[pl.ds(c * N + s * RPT, RPT)])

  return pl.kernel(
      body,
      mesh=_MESH,
      compiler_params=pltpu.CompilerParams(use_tc_tiling_on_sc=False),
      out_type=jax.ShapeDtypeStruct((NC * N, d), jnp.float32),
      scratch_types=(
          [pltpu.VMEM((nchunk, ch), jnp.int32),
           pltpu.VMEM((nchunk, ch), jnp.int32)]
          + [pltpu.VMEM((ch, d), jnp.float32) for _ in range(nbuf)]
          + [pltpu.VMEM_SHARED((N, d), jnp.float32)]
          + [pltpu.SemaphoreType.DMA for _ in range(nbuf)]
      ),
  ), ch


def _make_deg_kernel():
  def body(dst_hbm, out_hbm, dstv, ones, zb, acc):
    c = lax.axis_index("c")
    s = lax.axis_index("s")
    wid = s * NC + c

    pltpu.sync_copy(dst_hbm.at[wid], dstv)

    _zero_vmem_rows(zb, min(DEG_CH, RPT), 16)
    for r in range(RPT // DEG_CH):
      pltpu.sync_copy(zb, acc.at[pl.ds(s * RPT + r * DEG_CH, DEG_CH)])
    if RPT % DEG_CH:
      pltpu.sync_copy(zb.at[pl.ds(0, RPT % DEG_CH)],
                      acc.at[pl.ds(s * RPT + (RPT // DEG_CH) * DEG_CH,
                                   RPT % DEG_CH)])

    def setones(i, carry):
      ones[i, pl.ds(0, 16)] = jnp.ones((16,), jnp.float32)
      return carry
    lax.fori_loop(0, DEG_CH, setones, 0)
    plsc.subcore_barrier()

    def chunk(k, carry):
      pltpu.sync_copy(ones, acc.at[dstv.at[k]], add=True)
      return carry
    lax.fori_loop(0, EPW // DEG_CH, chunk, 0)

    plsc.subcore_barrier()
    pltpu.sync_copy(acc.at[pl.ds(s * RPT, RPT)],
                    out_hbm.at[pl.ds(c * N + s * RPT, RPT)])

  return pl.kernel(
      body,
      mesh=_MESH,
      compiler_params=pltpu.CompilerParams(use_tc_tiling_on_sc=False),
      out_type=jax.ShapeDtypeStruct((NC * N, 16), jnp.float32),
      scratch_types=[
          pltpu.VMEM((EPW // DEG_CH, DEG_CH), jnp.int32),
          pltpu.VMEM((DEG_CH, 16), jnp.float32),
          pltpu.VMEM((DEG_CH, 16), jnp.float32),
          pltpu.VMEM_SHARED((N, 16), jnp.float32),
      ],
  )


_agg_kernels = {}


def _agg(g, edge_index):
  d = g.shape[1]
  if d not in _agg_kernels:
    _agg_kernels[d] = _make_agg_kernel(d)
  k, ch = _agg_kernels[d]
  src3 = edge_index[0].reshape(NW, EPW // ch, ch)
  dst3 = edge_index[1].reshape(NW, EPW // ch, ch)
  return k(g, src3, dst3)


_deg_kernel = []


def _deg(dst2):
  if not _deg_kernel:
    _deg_kernel.append(_make_deg_kernel())
  return _deg_kernel[0](dst2)


_TCB = 2000
_TCG = N // _TCB


def _tc0(x, degp):
  def body(x_ref, d0_ref, d1_ref, dinv_ref, gp_ref):
    deg = d0_ref[:, :1] + d1_ref[:, :1] + 1.0
    dinv = lax.rsqrt(deg)
    dinv_ref[...] = dinv
    gp_ref[...] = x_ref[...] * dinv

  din = x.shape[1]
  return pl.pallas_call(
      body,
      grid=(_TCG,),
      in_specs=[
          pl.BlockSpec((_TCB, din), lambda i: (i, 0)),
          pl.BlockSpec((_TCB, 16), lambda i: (i, 0)),
          pl.BlockSpec((_TCB, 16), lambda i: (i + _TCG, 0)),
      ],
      out_specs=[
          pl.BlockSpec((_TCB, 1), lambda i: (i, 0)),
          pl.BlockSpec((_TCB, din), lambda i: (i, 0)),
      ],
      out_shape=[
          jax.ShapeDtypeStruct((N, 1), jnp.float32),
          jax.ShapeDtypeStruct((N, din), jnp.float32),
      ],
  )(x, degp, degp)


def _tc_layer(pp, gp, dinv, W, b, skip, agg_first, next_mode, next_w, want_x):
  da = gp.shape[1]
  dout = b.shape[0]
  b2 = b.reshape(1, dout)
  next_mm = next_mode == "mm"
  next_scale = next_mode == "scale"

  def body(*refs):
    idx = 0
    p0_ref = refs[idx]; idx += 1
    p1_ref = refs[idx]; idx += 1
    gp_ref = refs[idx]; idx += 1
    dinv_ref = refs[idx]; idx += 1
    if agg_first:
      w_ref = refs[idx]; idx += 1
    b_ref = refs[idx]; idx += 1
    if skip is not None:
      skip_ref = refs[idx]; idx += 1
    if next_mm:
      wn_ref = refs[idx]; idx += 1
    out_refs = refs[idx:]

    dinv_blk = dinv_ref[...]
    t = dinv_blk * (p0_ref[...] + p1_ref[...] + gp_ref[...])
    if agg_first:
      h = jnp.dot(t.astype(jnp.bfloat16), w_ref[...].astype(jnp.bfloat16),
                  preferred_element_type=jnp.float32) + b_ref[...]
    else:
      h = t + b_ref[...]
    if skip is not None:
      h = h + skip_ref[...]
    x = jnp.maximum(h, 0.0)
    oi = 0
    if want_x:
      out_refs[oi][...] = x
      oi += 1
    if next_scale:
      out_refs[oi][...] = dinv_blk * x
    elif next_mm:
      out_refs[oi][...] = dinv_blk * jnp.dot(
          x.astype(jnp.bfloat16), wn_ref[...].astype(jnp.bfloat16),
          preferred_element_type=jnp.float32)

  in_specs = [
      pl.BlockSpec((_TCB, da), lambda i: (i, 0)),
      pl.BlockSpec((_TCB, da), lambda i: (i + _TCG, 0)),
      pl.BlockSpec((_TCB, da), lambda i: (i, 0)),
      pl.BlockSpec((_TCB, 1), lambda i: (i, 0)),
  ]
  args = [pp, pp, gp, dinv]
  if agg_first:
    in_specs.append(pl.BlockSpec((da, dout), lambda i: (0, 0)))
    args.append(W)
  in_specs.append(pl.BlockSpec((1, dout), lambda i: (0, 0)))
  args.append(b2)
  if skip is not None:
    in_specs.append(pl.BlockSpec((_TCB, dout), lambda i: (i, 0)))
    args.append(skip)
  if next_mm:
    dnext = next_w.shape[1]
    in_specs.append(pl.BlockSpec((dout, dnext), lambda i: (0, 0)))
    args.append(next_w)

  out_specs = []
  out_shape = []
  if want_x:
    out_specs.append(pl.BlockSpec((_TCB, dout), lambda i: (i, 0)))
    out_shape.append(jax.ShapeDtypeStruct((N, dout), jnp.float32))
  if next_scale:
    out_specs.append(pl.BlockSpec((_TCB, dout), lambda i: (i, 0)))
    out_shape.append(jax.ShapeDtypeStruct((N, dout), jnp.float32))
  elif next_mm:
    dnext = next_w.shape[1]
    out_specs.append(pl.BlockSpec((_TCB, dnext), lambda i: (i, 0)))
    out_shape.append(jax.ShapeDtypeStruct((N, dnext), jnp.float32))

  res = pl.pallas_call(
      body,
      grid=(_TCG,),
      in_specs=in_specs,
      out_specs=out_specs,
      out_shape=out_shape,
  )(*args)
  x_out = res[0] if want_x else None
  gp_out = res[-1] if (next_scale or next_mm) else None
  return x_out, gp_out


DIMS = (128, 256, 128, 64, 32, 16, 32, 64, 128, 256, 128)
AGG_FIRST = tuple(DIMS[l - 1] <= DIMS[l] for l in range(1, 11))
SKIP_SRC = {6: 4, 7: 3, 8: 2, 9: 1}
WANT_X = {1, 2, 3, 4, 10}


def kernel(x, edge_index, W1, b1, W2, b2, W3, b3, W4, b4, W5, b5,
           W6, b6, W7, b7, W8, b8, W9, b9, W10, b10):
  Ws = (W1, W2, W3, W4, W5, W6, W7, W8, W9, W10)
  bs = (b1, b2, b3, b4, b5, b6, b7, b8, b9, b10)

  dst2 = edge_index[1].reshape(NW, EPW // DEG_CH, DEG_CH)

  degp = _deg(dst2)
  dinv, gp = _tc0(x, degp)

  saved = {}
  xl = None
  for l in range(1, 11):
    pp = _agg(gp, edge_index)
    skip = saved.get(SKIP_SRC.get(l))
    if l < 10:
      next_mode = "scale" if AGG_FIRST[l] else "mm"
      next_w = None if AGG_FIRST[l] else Ws[l]
    else:
      next_mode, next_w = None, None
    xl, gp = _tc_layer(pp, gp, dinv,
                       Ws[l - 1] if AGG_FIRST[l - 1] else None,
                       bs[l - 1], skip, AGG_FIRST[l - 1], next_mode, next_w,
                       l in WANT_X)
    if l in SKIP_SRC.values():
      saved[l] = xl
  return xl


# speedup vs baseline: 33.3289x; 1.0101x over previous
import jax
import jax.numpy as jnp
from jax import lax
from jax.experimental import pallas as pl
from jax.experimental.pallas import tpu as pltpu
from jax.experimental.pallas import tpu_sc as plsc

N = 10000
E = 320000
NC = 2
NS = 16
NW = NC * NS
EPW = E // NW
CH = 125
NCHUNK = EPW // CH
DEG_CH = 625
RPT = N // NS

_MESH = plsc.VectorSubcoreMesh(core_axis_name="c", subcore_axis_name="s")


def _zero_vmem_rows(buf, rows, width):
  def body(i, carry):
    for j in range(width // 16):
      buf[i, pl.ds(j * 16, 16)] = jnp.zeros((16,), jnp.float32)
    return carry
  lax.fori_loop(0, rows, body, 0)


def _make_agg_kernel(d):
  ch = {128: 100, 64: 250, 32: 500, 16: 625}[d]
  nbuf = 2
  nchunk = EPW // ch

  def body(g_hbm, src_hbm, dst_hbm, out_hbm, *rest):
    srcv, dstv = rest[0], rest[1]
    gbufs = rest[2:2 + nbuf]
    acc = rest[2 + nbuf]
    gsems = rest[3 + nbuf:]
    c = lax.axis_index("c")
    s = lax.axis_index("s")
    wid = s * NC + c

    pltpu.sync_copy(src_hbm.at[wid], srcv)
    pltpu.sync_copy(dst_hbm.at[wid], dstv)

    _zero_vmem_rows(gbufs[0], min(ch, RPT), d)
    for r in range(RPT // ch):
      pltpu.sync_copy(gbufs[0], acc.at[pl.ds(s * RPT + r * ch, ch)])
    rem = RPT - (RPT // ch) * ch
    if rem:
      pltpu.sync_copy(gbufs[0].at[pl.ds(0, rem)],
                      acc.at[pl.ds(s * RPT + (RPT // ch) * ch, rem)])
    plsc.subcore_barrier()

    for b in range(nbuf):
      pltpu.async_copy(g_hbm.at[srcv.at[b]], gbufs[b], gsems[b])

    def group(gi, carry):
      for b in range(nbuf):
        k = gi * nbuf + b
        pltpu.make_async_copy(g_hbm.at[srcv.at[k]], gbufs[b],
                              gsems[b]).wait()
        pltpu.sync_copy(gbufs[b], acc.at[dstv.at[k]], add=True)
        @pl.when(k + nbuf < nchunk)
        def _():
          pltpu.async_copy(g_hbm.at[srcv.at[k + nbuf]], gbufs[b], gsems[b])
      return carry
    lax.fori_loop(0, nchunk // nbuf, group, 0)

    plsc.subcore_barrier()
    pltpu.sync_copy(acc.at[pl.ds(s * RPT, RPT)],
                    out_hbm.at[pl.ds(c * N + s * RPT, RPT)])

  return pl.kernel(
      body,
      mesh=_MESH,
      compiler_params=pltpu.CompilerParams(use_tc_tiling_on_sc=False),
      out_type=jax.ShapeDtypeStruct((NC * N, d), jnp.float32),
      scratch_types=(
          [pltpu.VMEM((nchunk, ch), jnp.int32),
           pltpu.VMEM((nchunk, ch), jnp.int32)]
          + [pltpu.VMEM((ch, d), jnp.float32) for _ in range(nbuf)]
          + [pltpu.VMEM_SHARED((N, d), jnp.float32)]
          + [pltpu.SemaphoreType.DMA for _ in range(nbuf)]
      ),
  ), ch


def _make_deg_kernel():
  def body(dst_hbm, out_hbm, dstv, ones, zb, acc):
    c = lax.axis_index("c")
    s = lax.axis_index("s")
    wid = s * NC + c

    pltpu.sync_copy(dst_hbm.at[wid], dstv)

    _zero_vmem_rows(zb, min(DEG_CH, RPT), 16)
    for r in range(RPT // DEG_CH):
      pltpu.sync_copy(zb, acc.at[pl.ds(s * RPT + r * DEG_CH, DEG_CH)])
    if RPT % DEG_CH:
      pltpu.sync_copy(zb.at[pl.ds(0, RPT % DEG_CH)],
                      acc.at[pl.ds(s * RPT + (RPT // DEG_CH) * DEG_CH,
                                   RPT % DEG_CH)])

    def setones(i, carry):
      ones[i, pl.ds(0, 16)] = jnp.ones((16,), jnp.float32)
      return carry
    lax.fori_loop(0, DEG_CH, setones, 0)
    plsc.subcore_barrier()

    def chunk(k, carry):
      pltpu.sync_copy(ones, acc.at[dstv.at[k]], add=True)
      return carry
    lax.fori_loop(0, EPW // DEG_CH, chunk, 0)

    plsc.subcore_barrier()
    pltpu.sync_copy(acc.at[pl.ds(s * RPT, RPT)],
                    out_hbm.at[pl.ds(c * N + s * RPT, RPT)])

  return pl.kernel(
      body,
      mesh=_MESH,
      compiler_params=pltpu.CompilerParams(use_tc_tiling_on_sc=False),
      out_type=jax.ShapeDtypeStruct((NC * N, 16), jnp.float32),
      scratch_types=[
          pltpu.VMEM((EPW // DEG_CH, DEG_CH), jnp.int32),
          pltpu.VMEM((DEG_CH, 16), jnp.float32),
          pltpu.VMEM((DEG_CH, 16), jnp.float32),
          pltpu.VMEM_SHARED((N, 16), jnp.float32),
      ],
  )


_agg_kernels = {}


def _agg(g, edge_index):
  d = g.shape[1]
  if d not in _agg_kernels:
    _agg_kernels[d] = _make_agg_kernel(d)
  k, ch = _agg_kernels[d]
  src3 = edge_index[0].reshape(NW, EPW // ch, ch)
  dst3 = edge_index[1].reshape(NW, EPW // ch, ch)
  return k(g, src3, dst3)


_deg_kernel = []


def _deg(dst2):
  if not _deg_kernel:
    _deg_kernel.append(_make_deg_kernel())
  return _deg_kernel[0](dst2)


_TCB = 5000
_TCG = N // _TCB


def _tc0(x, degp):
  def body(x_ref, d0_ref, d1_ref, dinv_ref, gp_ref):
    deg = d0_ref[:, :1] + d1_ref[:, :1] + 1.0
    dinv = lax.rsqrt(deg)
    dinv_ref[...] = dinv
    gp_ref[...] = x_ref[...] * dinv

  din = x.shape[1]
  return pl.pallas_call(
      body,
      grid=(_TCG,),
      in_specs=[
          pl.BlockSpec((_TCB, din), lambda i: (i, 0)),
          pl.BlockSpec((_TCB, 16), lambda i: (i, 0)),
          pl.BlockSpec((_TCB, 16), lambda i: (i + _TCG, 0)),
      ],
      out_specs=[
          pl.BlockSpec((_TCB, 1), lambda i: (i, 0)),
          pl.BlockSpec((_TCB, din), lambda i: (i, 0)),
      ],
      out_shape=[
          jax.ShapeDtypeStruct((N, 1), jnp.float32),
          jax.ShapeDtypeStruct((N, din), jnp.float32),
      ],
  )(x, degp, degp)


def _tc_layer(pp, gp, dinv, W, b, skip, agg_first, next_mode, next_w, want_x):
  da = gp.shape[1]
  dout = b.shape[0]
  b2 = b.reshape(1, dout)
  next_mm = next_mode == "mm"
  next_scale = next_mode == "scale"

  def body(*refs):
    idx = 0
    p0_ref = refs[idx]; idx += 1
    p1_ref = refs[idx]; idx += 1
    gp_ref = refs[idx]; idx += 1
    dinv_ref = refs[idx]; idx += 1
    if agg_first:
      w_ref = refs[idx]; idx += 1
    b_ref = refs[idx]; idx += 1
    if skip is not None:
      skip_ref = refs[idx]; idx += 1
    if next_mm:
      wn_ref = refs[idx]; idx += 1
    out_refs = refs[idx:]

    dinv_blk = dinv_ref[...]
    t = dinv_blk * (p0_ref[...] + p1_ref[...] + gp_ref[...])
    if agg_first:
      h = jnp.dot(t.astype(jnp.bfloat16), w_ref[...].astype(jnp.bfloat16),
                  preferred_element_type=jnp.float32) + b_ref[...]
    else:
      h = t + b_ref[...]
    if skip is not None:
      h = h + skip_ref[...]
    x = jnp.maximum(h, 0.0)
    oi = 0
    if want_x:
      out_refs[oi][...] = x
      oi += 1
    if next_scale:
      out_refs[oi][...] = dinv_blk * x
    elif next_mm:
      out_refs[oi][...] = dinv_blk * jnp.dot(
          x.astype(jnp.bfloat16), wn_ref[...].astype(jnp.bfloat16),
          preferred_element_type=jnp.float32)

  in_specs = [
      pl.BlockSpec((_TCB, da), lambda i: (i, 0)),
      pl.BlockSpec((_TCB, da), lambda i: (i + _TCG, 0)),
      pl.BlockSpec((_TCB, da), lambda i: (i, 0)),
      pl.BlockSpec((_TCB, 1), lambda i: (i, 0)),
  ]
  args = [pp, pp, gp, dinv]
  if agg_first:
    in_specs.append(pl.BlockSpec((da, dout), lambda i: (0, 0)))
    args.append(W)
  in_specs.append(pl.BlockSpec((1, dout), lambda i: (0, 0)))
  args.append(b2)
  if skip is not None:
    in_specs.append(pl.BlockSpec((_TCB, dout), lambda i: (i, 0)))
    args.append(skip)
  if next_mm:
    dnext = next_w.shape[1]
    in_specs.append(pl.BlockSpec((dout, dnext), lambda i: (0, 0)))
    args.append(next_w)

  out_specs = []
  out_shape = []
  if want_x:
    out_specs.append(pl.BlockSpec((_TCB, dout), lambda i: (i, 0)))
    out_shape.append(jax.ShapeDtypeStruct((N, dout), jnp.float32))
  if next_scale:
    out_specs.append(pl.BlockSpec((_TCB, dout), lambda i: (i, 0)))
    out_shape.append(jax.ShapeDtypeStruct((N, dout), jnp.float32))
  elif next_mm:
    dnext = next_w.shape[1]
    out_specs.append(pl.BlockSpec((_TCB, dnext), lambda i: (i, 0)))
    out_shape.append(jax.ShapeDtypeStruct((N, dnext), jnp.float32))

  res = pl.pallas_call(
      body,
      grid=(_TCG,),
      in_specs=in_specs,
      out_specs=out_specs,
      out_shape=out_shape,
  )(*args)
  x_out = res[0] if want_x else None
  gp_out = res[-1] if (next_scale or next_mm) else None
  return x_out, gp_out


DIMS = (128, 256, 128, 64, 32, 16, 32, 64, 128, 256, 128)
AGG_FIRST = tuple(DIMS[l - 1] <= DIMS[l] for l in range(1, 11))
SKIP_SRC = {6: 4, 7: 3, 8: 2, 9: 1}
WANT_X = {1, 2, 3, 4, 10}


def kernel(x, edge_index, W1, b1, W2, b2, W3, b3, W4, b4, W5, b5,
           W6, b6, W7, b7, W8, b8, W9, b9, W10, b10):
  Ws = (W1, W2, W3, W4, W5, W6, W7, W8, W9, W10)
  bs = (b1, b2, b3, b4, b5, b6, b7, b8, b9, b10)

  dst2 = edge_index[1].reshape(NW, EPW // DEG_CH, DEG_CH)

  degp = _deg(dst2)
  dinv, gp = _tc0(x, degp)

  saved = {}
  xl = None
  for l in range(1, 11):
    pp = _agg(gp, edge_index)
    skip = saved.get(SKIP_SRC.get(l))
    if l < 10:
      next_mode = "scale" if AGG_FIRST[l] else "mm"
      next_w = None if AGG_FIRST[l] else Ws[l]
    else:
      next_mode, next_w = None, None
    xl, gp = _tc_layer(pp, gp, dinv,
                       Ws[l - 1] if AGG_FIRST[l - 1] else None,
                       bs[l - 1], skip, AGG_FIRST[l - 1], next_mode, next_w,
                       l in WANT_X)
    if l in SKIP_SRC.values():
      saved[l] = xl
  return xl
